# baseline (speedup 1.0000x reference)
import functools

import jax
import jax.numpy as jnp
from jax import lax
from jax.experimental import pallas as pl
from jax.experimental.pallas import tpu as pltpu
from jax.experimental.pallas import tpu_sc as plsc

N = 10000
E = 320000
IN_DIM = 128
H = 8
F1 = 16
F2 = 128

NC = 2
NS = 16
NW = NC * NS
EPW = E // NW
C = 80
NCHUNK = EPW // C
CD = 80
NCHUNK_D = EPW // CD
RPT = 624
TAIL = N - NS * RPT


def _dense_body(x_ref, w_ref, ab_ref, h_ref, tab_ref):
    h = jnp.dot(x_ref[...], w_ref[...], preferred_element_type=jnp.float32)
    h_ref[...] = h
    tab_ref[...] = jnp.dot(h, ab_ref[...], preferred_element_type=jnp.float32)


def _dense(x, w, ab, blk=400):
    n, din = x.shape
    dout = w.shape[1]
    return pl.pallas_call(
        _dense_body,
        grid=(n // blk,),
        in_specs=[
            pl.BlockSpec((blk, din), lambda i: (i, 0)),
            pl.BlockSpec((din, dout), lambda i: (0, 0)),
            pl.BlockSpec((dout, 16), lambda i: (0, 0)),
        ],
        out_specs=[
            pl.BlockSpec((blk, dout), lambda i: (i, 0)),
            pl.BlockSpec((blk, 16), lambda i: (i, 0)),
        ],
        out_shape=[
            jax.ShapeDtypeStruct((n, dout), jnp.float32),
            jax.ShapeDtypeStruct((n, 16), jnp.float32),
        ],
    )(x, w, ab)


def _dense2_body(a0_ref, a1_ref, b_ref, w_ref, ab_ref, h_ref, tab_ref):
    z = a0_ref[...] + a1_ref[...] + b_ref[...]
    z = jnp.where(z > 0, z, jnp.exp(z) - 1.0)
    h = jnp.dot(z, w_ref[...], preferred_element_type=jnp.float32)
    h_ref[...] = h
    tab_ref[...] = jnp.dot(h, ab_ref[...], preferred_element_type=jnp.float32)


def _dense2(a0, a1, b_row, w, ab, blk=400):
    n, din = a0.shape
    dout = w.shape[1]
    return pl.pallas_call(
        _dense2_body,
        grid=(n // blk,),
        in_specs=[
            pl.BlockSpec((blk, din), lambda i: (i, 0)),
            pl.BlockSpec((blk, din), lambda i: (i, 0)),
            pl.BlockSpec((1, din), lambda i: (0, 0)),
            pl.BlockSpec((din, dout), lambda i: (0, 0)),
            pl.BlockSpec((dout, 16), lambda i: (0, 0)),
        ],
        out_specs=[
            pl.BlockSpec((blk, dout), lambda i: (i, 0)),
            pl.BlockSpec((blk, 16), lambda i: (i, 0)),
        ],
        out_shape=[
            jax.ShapeDtypeStruct((n, dout), jnp.float32),
            jax.ShapeDtypeStruct((n, 16), jnp.float32),
        ],
    )(a0, a1, b_row, w, ab)


def _dinv_body(tab_ref, d0_ref, d1_ref, o_ref, *, scale):
    dinv = 1.0 / ((d0_ref[...] + d1_ref[...]) * scale)
    o_ref[...] = jnp.concatenate(
        [tab_ref[:, 8:16], dinv[:, 0:8]], axis=-1)


def _dinv(tab, d0, d1, scale):
    return pl.pallas_call(
        functools.partial(_dinv_body, scale=scale),
        out_shape=jax.ShapeDtypeStruct((N, 16), jnp.float32),
    )(tab, d0, d1)


def _final_body(a0_ref, a1_ref, b_ref, o_ref):
    o_ref[...] = a0_ref[...] + a1_ref[...] + b_ref[...]


def _final(a0, a1, b_row, blk=400):
    return pl.pallas_call(
        _final_body,
        grid=(N // blk,),
        in_specs=[
            pl.BlockSpec((blk, 128), lambda i: (i, 0)),
            pl.BlockSpec((blk, 128), lambda i: (i, 0)),
            pl.BlockSpec((1, 128), lambda i: (0, 0)),
        ],
        out_specs=pl.BlockSpec((blk, 128), lambda i: (i, 0)),
        out_shape=jax.ShapeDtypeStruct((N, 128), jnp.float32),
    )(a0, a1, b_row)


_MESH = dict(core_axis_name="c", subcore_axis_name="s", num_cores=NC,
             num_subcores=NS)
_SC_PARAMS = pltpu.CompilerParams(needs_layout_passes=False,
                                  use_tc_tiling_on_sc=False)


def _alpha_groups(stab, dtab, out_buf, c, combined):
    lanes = lax.iota(jnp.int32, 16)
    for g in range(c // 16):
        eidx = g * 16 + lanes
        for h in range(H):
            hcol = jnp.full((16,), h, jnp.int32)
            els = plsc.load_gather(stab, [eidx, hcol])
            if combined:
                erd = plsc.load_gather(dtab, [eidx, hcol])
            else:
                erd = plsc.load_gather(dtab, [eidx, hcol + 8])
            e = els + erd
            e = jnp.where(e > 0, e, 0.2 * e)
            v = jnp.exp(e)
            if combined:
                v = v * plsc.load_gather(dtab, [eidx, hcol + 8])
            plsc.store_scatter(out_buf, [eidx, hcol], v)


def _sc_den_kernel():
    mesh = plsc.VectorSubcoreMesh(**_MESH)

    @functools.partial(
        pl.kernel,
        out_type=jax.ShapeDtypeStruct((NC, N, 16), jnp.float32),
        mesh=mesh,
        compiler_params=_SC_PARAMS,
        scratch_types=[
            pltpu.VMEM((CD,), jnp.int32),
            pltpu.VMEM((CD,), jnp.int32),
            pltpu.VMEM((CD,), jnp.int32),
            pltpu.VMEM((CD,), jnp.int32),
            pltpu.VMEM((CD, 16), jnp.float32),
            pltpu.VMEM((CD, 16), jnp.float32),
            pltpu.VMEM((CD, 16), jnp.float32),
            pltpu.VMEM((CD, 16), jnp.float32),
            pltpu.VMEM((CD, 16), jnp.float32),
            pltpu.VMEM_SHARED((N, 16), jnp.float32),
            pltpu.SemaphoreType.DMA,
            pltpu.SemaphoreType.DMA,
            pltpu.SemaphoreType.DMA,
            pltpu.SemaphoreType.DMA,
        ],
    )
    def k(tab_hbm, src_hbm, dst_hbm, out_hbm,
          sv0, sv1, dv0, dv1, stab0, stab1, dtab0, dtab1, ee, den_sh,
          semi0, semi1, semt0, semt1):
        c = lax.axis_index("c")
        s = lax.axis_index("s")
        wid = s * NC + c
        r0 = s * RPT
        svs, dvs = (sv0, sv1), (dv0, dv1)
        stabs, dtabs = (stab0, stab1), (dtab0, dtab1)
        semi, semt = (semi0, semi1), (semt0, semt1)

        def zrow(i, carry):
            ee[i, :] = jnp.zeros((16,), jnp.float32)
            return carry
        lax.fori_loop(0, CD, zrow, None)
        for t in range(RPT // CD):
            pltpu.sync_copy(ee, den_sh.at[pl.ds(r0 + t * CD, CD)])
        rem = RPT % CD
        if rem:
            pltpu.sync_copy(ee.at[pl.ds(0, rem)],
                            den_sh.at[pl.ds(r0 + RPT - rem, rem)])

        @pl.when(s == NS - 1)
        def _zero_tail():
            pltpu.sync_copy(ee.at[pl.ds(0, TAIL)],
                            den_sh.at[pl.ds(NS * RPT, TAIL)])
        plsc.subcore_barrier()

        def idx_load(kk, b):
            base = wid * EPW + kk * CD
            pltpu.async_copy(src_hbm.at[pl.ds(base, CD)], svs[b], semi[b])
            pltpu.async_copy(dst_hbm.at[pl.ds(base, CD)], dvs[b], semi[b])

        def idx_wait(b):
            pltpu.make_async_copy(
                src_hbm.at[pl.ds(0, CD)], svs[b], semi[b]).wait()
            pltpu.make_async_copy(
                dst_hbm.at[pl.ds(0, CD)], dvs[b], semi[b]).wait()

        def tab_load(b):
            pltpu.async_copy(tab_hbm.at[svs[b]], stabs[b], semt[b])
            pltpu.async_copy(tab_hbm.at[dvs[b]], dtabs[b], semt[b])

        def tab_wait(b):
            pltpu.make_async_copy(
                tab_hbm.at[pl.ds(0, CD)], stabs[b], semt[b]).wait()
            pltpu.make_async_copy(
                tab_hbm.at[pl.ds(0, CD)], dtabs[b], semt[b]).wait()

        def compute(b):
            _alpha_groups(stabs[b], dtabs[b], ee, CD, False)
            pltpu.sync_copy(ee, den_sh.at[dvs[b]], add=True)

        idx_load(0, 0)
        idx_wait(0)
        tab_load(0)
        idx_load(1, 1)

        def pair(t, carry):
            kk0 = t * 2
            tab_wait(0)
            idx_wait(1)
            tab_load(1)
            compute(0)
            idx_load(kk0 + 2, 0)
            tab_wait(1)
            idx_wait(0)
            tab_load(0)
            compute(1)

            @pl.when(kk0 + 3 < NCHUNK_D)
            def _more():
                idx_load(kk0 + 3, 1)
            return carry
        lax.fori_loop(0, NCHUNK_D // 2, pair, None)
        tab_wait(0)
        compute(0)

        plsc.subcore_barrier()
        pltpu.sync_copy(den_sh.at[pl.ds(r0, RPT)],
                        out_hbm.at[c, pl.ds(r0, RPT)])

        @pl.when(s == NS - 1)
        def _copy_tail():
            pltpu.sync_copy(den_sh.at[pl.ds(NS * RPT, TAIL)],
                            out_hbm.at[c, pl.ds(NS * RPT, TAIL)])

    return k


def _sc_agg_kernel(hf, head_sum):
    mesh = plsc.VectorSubcoreMesh(**_MESH)

    @functools.partial(
        pl.kernel,
        out_type=jax.ShapeDtypeStruct((NC, N, 128), jnp.float32),
        mesh=mesh,
        compiler_params=_SC_PARAMS,
        scratch_types=[
            pltpu.VMEM((C,), jnp.int32),
            pltpu.VMEM((C,), jnp.int32),
            pltpu.VMEM((C,), jnp.int32),
            pltpu.VMEM((C,), jnp.int32),
            pltpu.VMEM((C, 16), jnp.float32),
            pltpu.VMEM((C, 16), jnp.float32),
            pltpu.VMEM((C, 16), jnp.float32),
            pltpu.VMEM((C, 16), jnp.float32),
            pltpu.VMEM((16, hf), jnp.float32),
            pltpu.VMEM((16, hf), jnp.float32),
            pltpu.VMEM((C, 8), jnp.float32),
            pltpu.VMEM((C, 128), jnp.float32),
            pltpu.VMEM_SHARED((N, 128), jnp.float32),
            pltpu.SemaphoreType.DMA,
            pltpu.SemaphoreType.DMA,
            pltpu.SemaphoreType.DMA,
            pltpu.SemaphoreType.DMA,
            pltpu.SemaphoreType.DMA,
            pltpu.SemaphoreType.DMA,
        ],
    )
    def k(tab_hbm, dinv_hbm, h_hbm, src_hbm, dst_hbm, out_hbm,
          sv0, sv1, dv0, dv1, stab0, stab1, dtab0, dtab1,
          hbuf0, hbuf1, alpha, msg, acc_sh,
          semi0, semi1, semt0, semt1, semh0, semh1):
        c = lax.axis_index("c")
        s = lax.axis_index("s")
        wid = s * NC + c
        r0 = s * RPT
        svs, dvs = (sv0, sv1), (dv0, dv1)
        stabs, dtabs = (stab0, stab1), (dtab0, dtab1)
        semi, semt = (semi0, semi1), (semt0, semt1)
        hbufs, semh = (hbuf0, hbuf1), (semh0, semh1)

        def zrow(i, carry):
            for j in range(8):
                msg[i, pl.ds(j * 16, 16)] = jnp.zeros((16,), jnp.float32)
            return carry
        lax.fori_loop(0, C, zrow, None)
        for t in range(RPT // C):
            pltpu.sync_copy(msg, acc_sh.at[pl.ds(r0 + t * C, C)])
        rem = RPT % C
        if rem:
            pltpu.sync_copy(msg.at[pl.ds(0, rem)],
                            acc_sh.at[pl.ds(r0 + RPT - rem, rem)])

        @pl.when(s == NS - 1)
        def _zero_tail():
            pltpu.sync_copy(msg.at[pl.ds(0, TAIL)],
                            acc_sh.at[pl.ds(NS * RPT, TAIL)])
        plsc.subcore_barrier()

        def idx_load(kk, b):
            base = wid * EPW + kk * C
            pltpu.async_copy(src_hbm.at[pl.ds(base, C)], svs[b], semi[b])
            pltpu.async_copy(dst_hbm.at[pl.ds(base, C)], dvs[b], semi[b])

        def idx_wait(b):
            pltpu.make_async_copy(
                src_hbm.at[pl.ds(0, C)], svs[b], semi[b]).wait()
            pltpu.make_async_copy(
                dst_hbm.at[pl.ds(0, C)], dvs[b], semi[b]).wait()

        def tab_load(b):
            pltpu.async_copy(tab_hbm.at[svs[b]], stabs[b], semt[b])
            pltpu.async_copy(dinv_hbm.at[dvs[b]], dtabs[b], semt[b])

        def tab_wait(b):
            pltpu.make_async_copy(
                tab_hbm.at[pl.ds(0, C)], stabs[b], semt[b]).wait()
            pltpu.make_async_copy(
                dinv_hbm.at[pl.ds(0, C)], dtabs[b], semt[b]).wait()

        def compute(b):
            _alpha_groups(stabs[b], dtabs[b], alpha, C, True)
            sv = svs[b]
            ngrp = C // 16
            cp = pltpu.async_copy(
                h_hbm.at[sv.at[pl.ds(0, 16)]], hbufs[0], semh[0])
            for g in range(ngrp):
                cp.wait()
                if g + 1 < ngrp:
                    cp = pltpu.async_copy(
                        h_hbm.at[sv.at[pl.ds((g + 1) * 16, 16)]],
                        hbufs[(g + 1) % 2], semh[(g + 1) % 2])
                hbuf = hbufs[g % 2]

                hcols = [jnp.full((16,), h, jnp.int32) for h in range(H)]

                def edge(ei, carry):
                    row = g * 16 + ei
                    rowv = jnp.full((16,), row, jnp.int32)
                    if head_sum:
                        acc = [jnp.zeros((16,), jnp.float32)
                               for _ in range(8)]
                        for h in range(H):
                            av = plsc.load_gather(alpha, [rowv, hcols[h]])
                            for j in range(8):
                                acc[j] = acc[j] + av * hbuf[
                                    ei, pl.ds(h * 128 + j * 16, 16)]
                        for j in range(8):
                            msg[row, pl.ds(j * 16, 16)] = acc[j]
                    else:
                        for h in range(H):
                            av = plsc.load_gather(alpha, [rowv, hcols[h]])
                            msg[row, pl.ds(h * 16, 16)] = (
                                av * hbuf[ei, pl.ds(h * 16, 16)])
                    return carry
                lax.fori_loop(0, 16, edge, None)
            pltpu.sync_copy(msg, acc_sh.at[dvs[b]], add=True)

        idx_load(0, 0)
        idx_wait(0)
        tab_load(0)
        idx_load(1, 1)

        def pair(t, carry):
            kk0 = t * 2
            tab_wait(0)
            idx_wait(1)
            tab_load(1)
            compute(0)
            idx_load(kk0 + 2, 0)
            tab_wait(1)
            idx_wait(0)
            tab_load(0)
            compute(1)

            @pl.when(kk0 + 3 < NCHUNK)
            def _more():
                idx_load(kk0 + 3, 1)
            return carry
        lax.fori_loop(0, NCHUNK // 2, pair, None)
        tab_wait(0)
        compute(0)

        plsc.subcore_barrier()
        pltpu.sync_copy(acc_sh.at[pl.ds(r0, RPT)],
                        out_hbm.at[c, pl.ds(r0, RPT)])

        @pl.when(s == NS - 1)
        def _copy_tail():
            pltpu.sync_copy(acc_sh.at[pl.ds(NS * RPT, TAIL)],
                            out_hbm.at[c, pl.ds(NS * RPT, TAIL)])

    return k


def _attn_mat(al, ar, f):
    eye = jnp.eye(H, dtype=jnp.float32)
    a = (al[:, :, None] * eye[:, None, :]).reshape(H * f, H)
    b = (ar[:, :, None] * eye[:, None, :]).reshape(H * f, H)
    return jnp.concatenate([a, b], axis=1)


def kernel(x, edge_index, W1, al1, ar1, b1, W2, al2, ar2, b2):
    src = edge_index[0]
    dst = edge_index[1]
    ab1 = _attn_mat(al1, ar1, F1)
    ab2 = _attn_mat(al2, ar2, F2)

    h1, tab1 = _dense(x, W1, ab1)
    den1 = _sc_den_kernel()(tab1, src, dst)
    dinv1 = _dinv(tab1, den1[0], den1[1], 1.0)
    acc1 = _sc_agg_kernel(H * F1, False)(tab1, dinv1, h1, src, dst)

    h2, tab2 = _dense2(acc1[0], acc1[1], b1.reshape(1, H * F1), W2, ab2)
    den2 = _sc_den_kernel()(tab2, src, dst)
    dinv2 = _dinv(tab2, den2[0], den2[1], float(H))
    acc2 = _sc_agg_kernel(H * F2, True)(tab2, dinv2, h2, src, dst)

    b2m = b2.reshape(H, F2).mean(axis=0).reshape(1, F2)
    return _final(acc2[0], acc2[1], b2m)


# speedup vs baseline: 1.0441x; 1.0441x over previous
import functools

import jax
import jax.numpy as jnp
from jax import lax
from jax.experimental import pallas as pl
from jax.experimental.pallas import tpu as pltpu
from jax.experimental.pallas import tpu_sc as plsc

N = 10000
E = 320000
IN_DIM = 128
H = 8
F1 = 16
F2 = 128

NC = 2
NS = 16
NW = NC * NS
EPW = E // NW
C = 80
NCHUNK = EPW // C
CD = 80
NCHUNK_D = EPW // CD
RPT = 624
TAIL = N - NS * RPT


def _dense_body(x_ref, w_ref, ab_ref, h_ref, tab_ref):
    h = jnp.dot(x_ref[...], w_ref[...], preferred_element_type=jnp.float32)
    h_ref[...] = h
    tab_ref[...] = jnp.dot(h, ab_ref[...], preferred_element_type=jnp.float32)


def _dense(x, w, ab, blk=400):
    n, din = x.shape
    dout = w.shape[1]
    return pl.pallas_call(
        _dense_body,
        grid=(n // blk,),
        in_specs=[
            pl.BlockSpec((blk, din), lambda i: (i, 0)),
            pl.BlockSpec((din, dout), lambda i: (0, 0)),
            pl.BlockSpec((dout, 16), lambda i: (0, 0)),
        ],
        out_specs=[
            pl.BlockSpec((blk, dout), lambda i: (i, 0)),
            pl.BlockSpec((blk, 16), lambda i: (i, 0)),
        ],
        out_shape=[
            jax.ShapeDtypeStruct((n, dout), jnp.float32),
            jax.ShapeDtypeStruct((n, 16), jnp.float32),
        ],
    )(x, w, ab)


def _dense2_body(a0_ref, a1_ref, b_ref, w_ref, ab_ref, h_ref, tab_ref):
    z = a0_ref[...] + a1_ref[...] + b_ref[...]
    z = jnp.where(z > 0, z, jnp.exp(z) - 1.0)
    h = jnp.dot(z, w_ref[...], preferred_element_type=jnp.float32)
    h_ref[...] = h
    tab_ref[...] = jnp.dot(h, ab_ref[...], preferred_element_type=jnp.float32)


def _dense2(a0, a1, b_row, w, ab, blk=400):
    n, din = a0.shape
    dout = w.shape[1]
    return pl.pallas_call(
        _dense2_body,
        grid=(n // blk,),
        in_specs=[
            pl.BlockSpec((blk, din), lambda i: (i, 0)),
            pl.BlockSpec((blk, din), lambda i: (i, 0)),
            pl.BlockSpec((1, din), lambda i: (0, 0)),
            pl.BlockSpec((din, dout), lambda i: (0, 0)),
            pl.BlockSpec((dout, 16), lambda i: (0, 0)),
        ],
        out_specs=[
            pl.BlockSpec((blk, dout), lambda i: (i, 0)),
            pl.BlockSpec((blk, 16), lambda i: (i, 0)),
        ],
        out_shape=[
            jax.ShapeDtypeStruct((n, dout), jnp.float32),
            jax.ShapeDtypeStruct((n, 16), jnp.float32),
        ],
    )(a0, a1, b_row, w, ab)


def _dinv_body(tab_ref, d0_ref, d1_ref, o_ref, *, scale):
    dinv = 1.0 / ((d0_ref[...] + d1_ref[...]) * scale)
    o_ref[...] = jnp.concatenate(
        [tab_ref[:, 8:16], dinv[:, 0:8]], axis=-1)


def _dinv(tab, d0, d1, scale):
    return pl.pallas_call(
        functools.partial(_dinv_body, scale=scale),
        out_shape=jax.ShapeDtypeStruct((N, 16), jnp.float32),
    )(tab, d0, d1)


def _final_body(a0_ref, a1_ref, b_ref, o_ref):
    o_ref[...] = a0_ref[...] + a1_ref[...] + b_ref[...]


def _final(a0, a1, b_row, blk=400):
    return pl.pallas_call(
        _final_body,
        grid=(N // blk,),
        in_specs=[
            pl.BlockSpec((blk, 128), lambda i: (i, 0)),
            pl.BlockSpec((blk, 128), lambda i: (i, 0)),
            pl.BlockSpec((1, 128), lambda i: (0, 0)),
        ],
        out_specs=pl.BlockSpec((blk, 128), lambda i: (i, 0)),
        out_shape=jax.ShapeDtypeStruct((N, 128), jnp.float32),
    )(a0, a1, b_row)


_MESH = dict(core_axis_name="c", subcore_axis_name="s", num_cores=NC,
             num_subcores=NS)
_SC_PARAMS = pltpu.CompilerParams(needs_layout_passes=False,
                                  use_tc_tiling_on_sc=False)


def _alpha_groups(stab, dtab, out_buf, c, combined):
    lanes = lax.iota(jnp.int32, 16)
    for g in range(c // 16):
        eidx = g * 16 + lanes
        for h in range(H):
            hcol = jnp.full((16,), h, jnp.int32)
            els = plsc.load_gather(stab, [eidx, hcol])
            if combined:
                erd = plsc.load_gather(dtab, [eidx, hcol])
            else:
                erd = plsc.load_gather(dtab, [eidx, hcol + 8])
            e = els + erd
            e = jnp.where(e > 0, e, 0.2 * e)
            v = jnp.exp(e)
            if combined:
                v = v * plsc.load_gather(dtab, [eidx, hcol + 8])
            plsc.store_scatter(out_buf, [eidx, hcol], v)


def _sc_den_kernel():
    mesh = plsc.VectorSubcoreMesh(**_MESH)

    @functools.partial(
        pl.kernel,
        out_type=jax.ShapeDtypeStruct((NC, N, 16), jnp.float32),
        mesh=mesh,
        compiler_params=_SC_PARAMS,
        scratch_types=[
            pltpu.VMEM((CD,), jnp.int32),
            pltpu.VMEM((CD,), jnp.int32),
            pltpu.VMEM((CD,), jnp.int32),
            pltpu.VMEM((CD,), jnp.int32),
            pltpu.VMEM((CD, 16), jnp.float32),
            pltpu.VMEM((CD, 16), jnp.float32),
            pltpu.VMEM((CD, 16), jnp.float32),
            pltpu.VMEM((CD, 16), jnp.float32),
            pltpu.VMEM((CD, 16), jnp.float32),
            pltpu.VMEM_SHARED((N, 16), jnp.float32),
            pltpu.SemaphoreType.DMA,
            pltpu.SemaphoreType.DMA,
            pltpu.SemaphoreType.DMA,
            pltpu.SemaphoreType.DMA,
        ],
    )
    def k(tab_hbm, src_hbm, dst_hbm, out_hbm,
          sv0, sv1, dv0, dv1, stab0, stab1, dtab0, dtab1, ee, den_sh,
          semi0, semi1, semt0, semt1):
        c = lax.axis_index("c")
        s = lax.axis_index("s")
        wid = s * NC + c
        r0 = s * RPT
        svs, dvs = (sv0, sv1), (dv0, dv1)
        stabs, dtabs = (stab0, stab1), (dtab0, dtab1)
        semi, semt = (semi0, semi1), (semt0, semt1)

        def zrow(i, carry):
            ee[i, :] = jnp.zeros((16,), jnp.float32)
            return carry
        lax.fori_loop(0, CD, zrow, None)
        for t in range(RPT // CD):
            pltpu.sync_copy(ee, den_sh.at[pl.ds(r0 + t * CD, CD)])
        rem = RPT % CD
        if rem:
            pltpu.sync_copy(ee.at[pl.ds(0, rem)],
                            den_sh.at[pl.ds(r0 + RPT - rem, rem)])

        @pl.when(s == NS - 1)
        def _zero_tail():
            pltpu.sync_copy(ee.at[pl.ds(0, TAIL)],
                            den_sh.at[pl.ds(NS * RPT, TAIL)])
        plsc.subcore_barrier()

        def idx_load(kk, b):
            base = wid * EPW + kk * CD
            pltpu.async_copy(src_hbm.at[pl.ds(base, CD)], svs[b], semi[b])
            pltpu.async_copy(dst_hbm.at[pl.ds(base, CD)], dvs[b], semi[b])

        def idx_wait(b):
            pltpu.make_async_copy(
                src_hbm.at[pl.ds(0, CD)], svs[b], semi[b]).wait()
            pltpu.make_async_copy(
                dst_hbm.at[pl.ds(0, CD)], dvs[b], semi[b]).wait()

        def tab_load(b):
            pltpu.async_copy(tab_hbm.at[svs[b]], stabs[b], semt[b])
            pltpu.async_copy(tab_hbm.at[dvs[b]], dtabs[b], semt[b])

        def tab_wait(b):
            pltpu.make_async_copy(
                tab_hbm.at[pl.ds(0, CD)], stabs[b], semt[b]).wait()
            pltpu.make_async_copy(
                tab_hbm.at[pl.ds(0, CD)], dtabs[b], semt[b]).wait()

        def compute(b):
            _alpha_groups(stabs[b], dtabs[b], ee, CD, False)
            pltpu.sync_copy(ee, den_sh.at[dvs[b]], add=True)

        idx_load(0, 0)
        idx_wait(0)
        tab_load(0)
        idx_load(1, 1)

        def pair(t, carry):
            kk0 = t * 2
            tab_wait(0)
            idx_wait(1)
            tab_load(1)
            compute(0)
            idx_load(kk0 + 2, 0)
            tab_wait(1)
            idx_wait(0)
            tab_load(0)
            compute(1)

            @pl.when(kk0 + 3 < NCHUNK_D)
            def _more():
                idx_load(kk0 + 3, 1)
            return carry
        lax.fori_loop(0, NCHUNK_D // 2, pair, None)
        tab_wait(0)
        compute(0)

        plsc.subcore_barrier()
        pltpu.sync_copy(den_sh.at[pl.ds(r0, RPT)],
                        out_hbm.at[c, pl.ds(r0, RPT)])

        @pl.when(s == NS - 1)
        def _copy_tail():
            pltpu.sync_copy(den_sh.at[pl.ds(NS * RPT, TAIL)],
                            out_hbm.at[c, pl.ds(NS * RPT, TAIL)])

    return k


def _sc_agg_kernel(hf, head_sum):
    mesh = plsc.VectorSubcoreMesh(**_MESH)

    @functools.partial(
        pl.kernel,
        out_type=jax.ShapeDtypeStruct((NC, N, 128), jnp.float32),
        mesh=mesh,
        compiler_params=_SC_PARAMS,
        scratch_types=[
            pltpu.VMEM((C,), jnp.int32),
            pltpu.VMEM((C,), jnp.int32),
            pltpu.VMEM((C,), jnp.int32),
            pltpu.VMEM((C,), jnp.int32),
            pltpu.VMEM((C, 16), jnp.float32),
            pltpu.VMEM((C, 16), jnp.float32),
            pltpu.VMEM((C, 16), jnp.float32),
            pltpu.VMEM((C, 16), jnp.float32),
            pltpu.VMEM((16, hf) if head_sum else (C, hf), jnp.float32),
            pltpu.VMEM((16, hf) if head_sum else (C, hf), jnp.float32),
            pltpu.VMEM((C, 8), jnp.float32),
            pltpu.VMEM((C, 128), jnp.float32),
            pltpu.VMEM_SHARED((N, 128), jnp.float32),
            pltpu.SemaphoreType.DMA,
            pltpu.SemaphoreType.DMA,
            pltpu.SemaphoreType.DMA,
            pltpu.SemaphoreType.DMA,
            pltpu.SemaphoreType.DMA,
            pltpu.SemaphoreType.DMA,
        ],
    )
    def k(tab_hbm, dinv_hbm, h_hbm, src_hbm, dst_hbm, out_hbm,
          sv0, sv1, dv0, dv1, stab0, stab1, dtab0, dtab1,
          hbuf0, hbuf1, alpha, msg, acc_sh,
          semi0, semi1, semt0, semt1, semh0, semh1):
        c = lax.axis_index("c")
        s = lax.axis_index("s")
        wid = s * NC + c
        r0 = s * RPT
        svs, dvs = (sv0, sv1), (dv0, dv1)
        stabs, dtabs = (stab0, stab1), (dtab0, dtab1)
        semi, semt = (semi0, semi1), (semt0, semt1)
        hbufs, semh = (hbuf0, hbuf1), (semh0, semh1)

        def zrow(i, carry):
            for j in range(8):
                msg[i, pl.ds(j * 16, 16)] = jnp.zeros((16,), jnp.float32)
            return carry
        lax.fori_loop(0, C, zrow, None)
        for t in range(RPT // C):
            pltpu.sync_copy(msg, acc_sh.at[pl.ds(r0 + t * C, C)])
        rem = RPT % C
        if rem:
            pltpu.sync_copy(msg.at[pl.ds(0, rem)],
                            acc_sh.at[pl.ds(r0 + RPT - rem, rem)])

        @pl.when(s == NS - 1)
        def _zero_tail():
            pltpu.sync_copy(msg.at[pl.ds(0, TAIL)],
                            acc_sh.at[pl.ds(NS * RPT, TAIL)])
        plsc.subcore_barrier()

        def idx_load(kk, b):
            base = wid * EPW + kk * C
            pltpu.async_copy(src_hbm.at[pl.ds(base, C)], svs[b], semi[b])
            pltpu.async_copy(dst_hbm.at[pl.ds(base, C)], dvs[b], semi[b])

        def idx_wait(b):
            pltpu.make_async_copy(
                src_hbm.at[pl.ds(0, C)], svs[b], semi[b]).wait()
            pltpu.make_async_copy(
                dst_hbm.at[pl.ds(0, C)], dvs[b], semi[b]).wait()

        def tab_load(b):
            pltpu.async_copy(tab_hbm.at[svs[b]], stabs[b], semt[b])
            pltpu.async_copy(dinv_hbm.at[dvs[b]], dtabs[b], semt[b])
            if not head_sum:
                pltpu.async_copy(h_hbm.at[svs[b]], hbufs[b], semh[b])

        def tab_wait(b):
            pltpu.make_async_copy(
                tab_hbm.at[pl.ds(0, C)], stabs[b], semt[b]).wait()
            pltpu.make_async_copy(
                dinv_hbm.at[pl.ds(0, C)], dtabs[b], semt[b]).wait()
            if not head_sum:
                pltpu.make_async_copy(
                    h_hbm.at[pl.ds(0, C)], hbufs[b], semh[b]).wait()

        hcols = [jnp.full((16,), h, jnp.int32) for h in range(H)]

        def compute(b):
            _alpha_groups(stabs[b], dtabs[b], alpha, C, True)
            sv = svs[b]
            if not head_sum:
                hbuf = hbufs[b]

                def edge1(ei, carry):
                    rowv = jnp.full((16,), ei, jnp.int32)
                    for h in range(H):
                        av = plsc.load_gather(alpha, [rowv, hcols[h]])
                        msg[ei, pl.ds(h * 16, 16)] = (
                            av * hbuf[ei, pl.ds(h * 16, 16)])
                    return carry
                lax.fori_loop(0, C, edge1, None)
            else:
                ngrp = C // 16
                cp = pltpu.async_copy(
                    h_hbm.at[sv.at[pl.ds(0, 16)]], hbufs[0], semh[0])
                for g in range(ngrp):
                    cp.wait()
                    if g + 1 < ngrp:
                        cp = pltpu.async_copy(
                            h_hbm.at[sv.at[pl.ds((g + 1) * 16, 16)]],
                            hbufs[(g + 1) % 2], semh[(g + 1) % 2])
                    hbuf = hbufs[g % 2]

                    def edge(ei, carry):
                        row = g * 16 + ei
                        rowv = jnp.full((16,), row, jnp.int32)
                        acc = [jnp.zeros((16,), jnp.float32)
                               for _ in range(8)]
                        for h in range(H):
                            av = plsc.load_gather(alpha, [rowv, hcols[h]])
                            for j in range(8):
                                acc[j] = acc[j] + av * hbuf[
                                    ei, pl.ds(h * 128 + j * 16, 16)]
                        for j in range(8):
                            msg[row, pl.ds(j * 16, 16)] = acc[j]
                        return carry
                    lax.fori_loop(0, 16, edge, None)
            pltpu.sync_copy(msg, acc_sh.at[dvs[b]], add=True)

        idx_load(0, 0)
        idx_wait(0)
        tab_load(0)
        idx_load(1, 1)

        def pair(t, carry):
            kk0 = t * 2
            tab_wait(0)
            idx_wait(1)
            tab_load(1)
            compute(0)
            idx_load(kk0 + 2, 0)
            tab_wait(1)
            idx_wait(0)
            tab_load(0)
            compute(1)

            @pl.when(kk0 + 3 < NCHUNK)
            def _more():
                idx_load(kk0 + 3, 1)
            return carry
        lax.fori_loop(0, NCHUNK // 2, pair, None)
        tab_wait(0)
        compute(0)

        plsc.subcore_barrier()
        pltpu.sync_copy(acc_sh.at[pl.ds(r0, RPT)],
                        out_hbm.at[c, pl.ds(r0, RPT)])

        @pl.when(s == NS - 1)
        def _copy_tail():
            pltpu.sync_copy(acc_sh.at[pl.ds(NS * RPT, TAIL)],
                            out_hbm.at[c, pl.ds(NS * RPT, TAIL)])

    return k


def _attn_mat(al, ar, f):
    eye = jnp.eye(H, dtype=jnp.float32)
    a = (al[:, :, None] * eye[:, None, :]).reshape(H * f, H)
    b = (ar[:, :, None] * eye[:, None, :]).reshape(H * f, H)
    return jnp.concatenate([a, b], axis=1)


def kernel(x, edge_index, W1, al1, ar1, b1, W2, al2, ar2, b2):
    src = edge_index[0]
    dst = edge_index[1]
    ab1 = _attn_mat(al1, ar1, F1)
    ab2 = _attn_mat(al2, ar2, F2)

    h1, tab1 = _dense(x, W1, ab1)
    den1 = _sc_den_kernel()(tab1, src, dst)
    dinv1 = _dinv(tab1, den1[0], den1[1], 1.0)
    acc1 = _sc_agg_kernel(H * F1, False)(tab1, dinv1, h1, src, dst)

    h2, tab2 = _dense2(acc1[0], acc1[1], b1.reshape(1, H * F1), W2, ab2)
    den2 = _sc_den_kernel()(tab2, src, dst)
    dinv2 = _dinv(tab2, den2[0], den2[1], float(H))
    acc2 = _sc_agg_kernel(H * F2, True)(tab2, dinv2, h2, src, dst)

    b2m = b2.reshape(H, F2).mean(axis=0).reshape(1, F2)
    return _final(acc2[0], acc2[1], b2m)
